# baseline (speedup 1.0000x reference)
import jax
import jax.numpy as jnp
from jax import lax
from jax.experimental import pallas as pl
from jax.experimental.pallas import tpu as pltpu
from jax.experimental.pallas import tpu_sc as plsc

F = 26
H = 100000
K = 16
B = 4096

NC = 2
NS = 16
L = 16
NW = NC * NS
C = B // NW
NPAIR = (F * (F - 1)) // 2

HB = 1000
NHB = H // HB
NCHUNK = F * NHB
CE = HB * K

_MESH = plsc.VectorSubcoreMesh(core_axis_name="c", subcore_axis_name="s",
                               num_cores=NC, num_subcores=NS)
_PARAMS = pltpu.CompilerParams(needs_layout_passes=False,
                               use_tc_tiling_on_sc=False)


def _repack_body(vt_hbm, vc_hbm, col0, col1, rows0, rows1,
                 semi0, semi1, semo0, semo1):
    cid = lax.axis_index("c")
    sid = lax.axis_index("s")
    wid = sid * NC + cid
    cnt = jnp.where(wid < NCHUNK - NW * (NCHUNK // NW), NCHUNK // NW + 1,
                    NCHUNK // NW)

    lane = lax.iota(jnp.int32, L)
    gidx = lane * HB

    def _chunk(t):
        return wid + NW * t

    def _issue_in(c, col, semi):
        f = c // NHB
        h0 = (c % NHB) * HB
        for k in range(K):
            pltpu.async_copy(vt_hbm.at[f, k, pl.ds(h0, HB)],
                             col.at[pl.ds(k * HB, HB)], semi)

    def _wait_in(col, semi):
        for k in range(K):
            pltpu.make_async_copy(vt_hbm.at[0, 0, pl.ds(0, HB)],
                                  col.at[pl.ds(k * HB, HB)], semi).wait()

    def _transpose(col, rows):
        def _row(h, _):
            r = plsc.load_gather(col, [gidx + h])
            rows[pl.ds(h * K, K)] = r
            return 0
        lax.fori_loop(0, HB, _row, 0, unroll=16)

    def _issue_out(c, rows, semo):
        f = c // NHB
        h0 = (c % NHB) * HB
        base = (f * H + h0) * K
        pltpu.async_copy(rows, vc_hbm.at[pl.ds(base, CE)], semo)

    def _wait_out(rows, semo):
        pltpu.make_async_copy(rows, vc_hbm.at[pl.ds(0, CE)], semo).wait()

    _issue_in(_chunk(0), col0, semi0)
    _issue_in(_chunk(1), col1, semi1)

    def _step(t, _):
        c = _chunk(2 * t)
        _wait_in(col0, semi0)
        _transpose(col0, rows0)

        @pl.when(t > 0)
        def _():
            _wait_out(rows0, semo0)

        _issue_out(c, rows0, semo0)

        @pl.when(2 * t + 2 < cnt)
        def _():
            _issue_in(c + 2 * NW, col0, semi0)

        _wait_in(col1, semi1)
        _transpose(col1, rows1)

        @pl.when(t > 0)
        def _():
            _wait_out(rows1, semo1)

        _issue_out(c + NW, rows1, semo1)

        @pl.when(2 * t + 3 < cnt)
        def _():
            _issue_in(c + 3 * NW, col1, semi1)
        return 0

    lax.fori_loop(0, cnt // 2, _step, 0)

    @pl.when(cnt % 2 == 1)
    def _():
        _wait_in(col0, semi0)
        _transpose(col0, rows0)
        _wait_out(rows0, semo0)
        _issue_out(_chunk(cnt - 1), rows0, semo0)

    _wait_out(rows0, semo0)
    _wait_out(rows1, semo1)


def _ffm_body(x_hbm, v_hbm, w1_hbm, w0_hbm, out_hbm,
              xv, w1v, w0v, idx0, idx1, rows0, rows1, acc, lin, res,
              sem0, sem1, semw):
    cid = lax.axis_index("c")
    sid = lax.axis_index("s")
    wid = sid * NC + cid
    base = wid * C

    pltpu.sync_copy(x_hbm.at[:, pl.ds(base, C)], xv)
    w1copy = pltpu.async_copy(w1_hbm, w1v, semw)
    pltpu.sync_copy(w0_hbm, w0v)

    zero = jnp.zeros((L,), jnp.float32)
    def _z(b, _):
        acc[b] = zero
        return 0
    lax.fori_loop(0, C, _z, 0, unroll=8)

    lane = lax.iota(jnp.int32, L)

    def _build(i, j, idx):
        offa = jnp.full((L,), j * H, jnp.int32)
        offb = jnp.full((L,), i * H, jnp.int32)
        for k in range(C // L):
            idx[0, pl.ds(k * L, L)] = xv[i, pl.ds(k * L, L)] + offa
            idx[1, pl.ds(k * L, L)] = xv[j, pl.ds(k * L, L)] + offb

    def _issue(idx, rows, sem):
        pltpu.async_copy(v_hbm.at[idx.at[0]], rows.at[0], sem)
        pltpu.async_copy(v_hbm.at[idx.at[1]], rows.at[1], sem)

    def _wait(idx, rows, sem):
        pltpu.make_async_copy(v_hbm.at[idx.at[0]], rows.at[0], sem).wait()
        pltpu.make_async_copy(v_hbm.at[idx.at[1]], rows.at[1], sem).wait()

    def _mac_rows(rows):
        def _mac(b, _):
            plsc.addupdate(acc.at[b], rows[0, b] * rows[1, b])
            return 0
        lax.fori_loop(0, C, _mac, 0, unroll=8)

    def _adv(i, j):
        last = j == (F - 1)
        i2 = jnp.where(last, i + 1, i)
        j2 = jnp.where(last, i2 + 1, j + 1)
        return i2, j2

    _build(0, 1, idx0)
    _issue(idx0, rows0, sem0)

    def _step(t, carry):
        i1, j1 = carry
        _build(i1, j1, idx1)
        _issue(idx1, rows1, sem1)
        _wait(idx0, rows0, sem0)
        _mac_rows(rows0)
        i2, j2 = _adv(i1, j1)
        _build(i2, j2, idx0)
        _issue(idx0, rows0, sem0)
        _wait(idx1, rows1, sem1)
        _mac_rows(rows1)
        return _adv(i2, j2)

    lax.fori_loop(0, (NPAIR - 1) // 2, _step, (jnp.int32(0), jnp.int32(2)))
    _wait(idx0, rows0, sem0)
    _mac_rows(rows0)

    w1copy.wait()
    for k in range(C // L):
        lin[pl.ds(k * L, L)] = zero

    def _lin_f(f, _):
        for k in range(C // L):
            idx = xv[f, pl.ds(k * L, L)]
            w = plsc.load_gather(w1v, [idx])
            plsc.addupdate(lin.at[pl.ds(k * L, L)], w)
        return 0
    lax.fori_loop(0, F, _lin_f, 0)

    w0vec = w0v[...]
    for g in range(C // L):
        t = jnp.zeros((L,), jnp.float32)
        for m in range(L):
            s = jnp.sum(acc[g * L + m], axis=0)
            t = jnp.where(lane == m, s, t)
        z = lin[pl.ds(g * L, L)] + t + w0vec
        res[pl.ds(g * L, L)] = 1.0 / (1.0 + jnp.exp(-z))
    pltpu.sync_copy(res, out_hbm.at[pl.ds(base, C)])


def kernel(x, v, w1, w0):
    x = x.astype(jnp.int32)
    w1f = w1.reshape(H)
    w0v = jnp.broadcast_to(w0.astype(jnp.float32), (L,))
    vt = jnp.transpose(v, (0, 2, 1))

    repack = pl.kernel(
        _repack_body,
        out_type=jax.ShapeDtypeStruct((F * H * K,), jnp.float32),
        mesh=_MESH,
        compiler_params=_PARAMS,
        scratch_types=[
            pltpu.VMEM((CE,), jnp.float32),
            pltpu.VMEM((CE,), jnp.float32),
            pltpu.VMEM((CE,), jnp.float32),
            pltpu.VMEM((CE,), jnp.float32),
            pltpu.SemaphoreType.DMA,
            pltpu.SemaphoreType.DMA,
            pltpu.SemaphoreType.DMA,
            pltpu.SemaphoreType.DMA,
        ],
    )
    vc = repack(vt).reshape(F * H, K)

    ffm = pl.kernel(
        _ffm_body,
        out_type=jax.ShapeDtypeStruct((B,), jnp.float32),
        mesh=_MESH,
        compiler_params=_PARAMS,
        scratch_types=[
            pltpu.VMEM((F, C), jnp.int32),
            pltpu.VMEM((H,), jnp.float32),
            pltpu.VMEM((L,), jnp.float32),
            pltpu.VMEM((2, C), jnp.int32),
            pltpu.VMEM((2, C), jnp.int32),
            pltpu.VMEM((2, C, K), jnp.float32),
            pltpu.VMEM((2, C, K), jnp.float32),
            pltpu.VMEM((C, K), jnp.float32),
            pltpu.VMEM((C,), jnp.float32),
            pltpu.VMEM((C,), jnp.float32),
            pltpu.SemaphoreType.DMA,
            pltpu.SemaphoreType.DMA,
            pltpu.SemaphoreType.DMA,
        ],
    )
    return ffm(x, vc, w1f, w0v)


# speedup vs baseline: 1.0036x; 1.0036x over previous
import jax
import jax.numpy as jnp
from jax import lax
from jax.experimental import pallas as pl
from jax.experimental.pallas import tpu as pltpu
from jax.experimental.pallas import tpu_sc as plsc

F = 26
H = 100000
K = 16
B = 4096

NC = 2
NS = 16
L = 16
NW = NC * NS
C = B // NW
NPAIR = (F * (F - 1)) // 2

HB = 1000
NHB = H // HB
CE = HB * K
FSPLIT = 13

_MESH = plsc.VectorSubcoreMesh(core_axis_name="c", subcore_axis_name="s",
                               num_cores=NC, num_subcores=NS)
_PARAMS = pltpu.CompilerParams(needs_layout_passes=False,
                               use_tc_tiling_on_sc=False)


def _repack_body(vt_hbm, vc_hbm, col0, col1, rows0, rows1,
                 semi0, semi1, semo0, semo1, nchunk):
    cid = lax.axis_index("c")
    sid = lax.axis_index("s")
    wid = sid * NC + cid
    cnt = jnp.where(wid < nchunk - NW * (nchunk // NW), nchunk // NW + 1,
                    nchunk // NW)

    lane = lax.iota(jnp.int32, L)
    gidx = lane * HB

    def _chunk(t):
        return wid + NW * t

    def _issue_in(c, col, semi):
        f = c // NHB
        h0 = (c % NHB) * HB
        for k in range(K):
            pltpu.async_copy(vt_hbm.at[f, k, pl.ds(h0, HB)],
                             col.at[pl.ds(k * HB, HB)], semi)

    def _wait_in(col, semi):
        for k in range(K):
            pltpu.make_async_copy(vt_hbm.at[0, 0, pl.ds(0, HB)],
                                  col.at[pl.ds(k * HB, HB)], semi).wait()

    def _transpose(col, rows):
        def _row(h, _):
            r = plsc.load_gather(col, [gidx + h])
            rows[pl.ds(h * K, K)] = r
            return 0
        lax.fori_loop(0, HB, _row, 0, unroll=16)

    def _issue_out(c, rows, semo):
        f = c // NHB
        h0 = (c % NHB) * HB
        base = (f * H + h0) * K
        pltpu.async_copy(rows, vc_hbm.at[pl.ds(base, CE)], semo)

    def _wait_out(rows, semo):
        pltpu.make_async_copy(rows, vc_hbm.at[pl.ds(0, CE)], semo).wait()

    _issue_in(_chunk(0), col0, semi0)
    _issue_in(_chunk(1), col1, semi1)

    def _step(t, _):
        c = _chunk(2 * t)
        _wait_in(col0, semi0)
        _transpose(col0, rows0)

        @pl.when(t > 0)
        def _():
            _wait_out(rows0, semo0)

        _issue_out(c, rows0, semo0)

        @pl.when(2 * t + 2 < cnt)
        def _():
            _issue_in(c + 2 * NW, col0, semi0)

        _wait_in(col1, semi1)
        _transpose(col1, rows1)

        @pl.when(t > 0)
        def _():
            _wait_out(rows1, semo1)

        _issue_out(c + NW, rows1, semo1)

        @pl.when(2 * t + 3 < cnt)
        def _():
            _issue_in(c + 3 * NW, col1, semi1)
        return 0

    lax.fori_loop(0, cnt // 2, _step, 0)

    @pl.when(cnt % 2 == 1)
    def _():
        _wait_in(col0, semi0)
        _transpose(col0, rows0)
        _wait_out(rows0, semo0)
        _issue_out(_chunk(cnt - 1), rows0, semo0)

    _wait_out(rows0, semo0)
    _wait_out(rows1, semo1)


def _ffm_body(x_hbm, v1_hbm, v2_hbm, w1_hbm, w0_hbm, out_hbm,
              xv, w1v, w0v, idx0, idx1, rows0, rows1, acc, lin, res,
              sem0, sem1, semw):
    cid = lax.axis_index("c")
    sid = lax.axis_index("s")
    wid = sid * NC + cid
    base = wid * C

    pltpu.sync_copy(x_hbm.at[:, pl.ds(base, C)], xv)
    w1copy = pltpu.async_copy(w1_hbm, w1v, semw)
    pltpu.sync_copy(w0_hbm, w0v)

    zero = jnp.zeros((L,), jnp.float32)
    def _z(b, _):
        acc[b] = zero
        return 0
    lax.fori_loop(0, C, _z, 0, unroll=8)

    lane = lax.iota(jnp.int32, L)

    def _build(i, j, idx):
        ja = jnp.where(j >= FSPLIT, j - FSPLIT, j)
        ib = jnp.where(i >= FSPLIT, i - FSPLIT, i)
        offa = jnp.full((L,), ja * H, jnp.int32)
        offb = jnp.full((L,), ib * H, jnp.int32)
        for k in range(C // L):
            idx[0, pl.ds(k * L, L)] = xv[i, pl.ds(k * L, L)] + offa
            idx[1, pl.ds(k * L, L)] = xv[j, pl.ds(k * L, L)] + offb

    def _issue(i, j, idx, rows, sem):
        @pl.when(j < FSPLIT)
        def _():
            pltpu.async_copy(v1_hbm.at[idx.at[0]], rows.at[0], sem)

        @pl.when(j >= FSPLIT)
        def _():
            pltpu.async_copy(v2_hbm.at[idx.at[0]], rows.at[0], sem)

        @pl.when(i < FSPLIT)
        def _():
            pltpu.async_copy(v1_hbm.at[idx.at[1]], rows.at[1], sem)

        @pl.when(i >= FSPLIT)
        def _():
            pltpu.async_copy(v2_hbm.at[idx.at[1]], rows.at[1], sem)

    def _wait(idx, rows, sem):
        pltpu.make_async_copy(v1_hbm.at[idx.at[0]], rows.at[0], sem).wait()
        pltpu.make_async_copy(v1_hbm.at[idx.at[1]], rows.at[1], sem).wait()

    def _mac_rows(rows):
        def _mac(b, _):
            plsc.addupdate(acc.at[b], rows[0, b] * rows[1, b])
            return 0
        lax.fori_loop(0, C, _mac, 0, unroll=8)

    def _adv(i, j):
        last = j == (F - 1)
        i2 = jnp.where(last, i + 1, i)
        j2 = jnp.where(last, i2 + 1, j + 1)
        return i2, j2

    i0 = jnp.int32(0)
    j0 = jnp.int32(1)
    _build(i0, j0, idx0)
    _issue(i0, j0, idx0, rows0, sem0)

    def _step(t, carry):
        i1, j1 = carry
        _build(i1, j1, idx1)
        _issue(i1, j1, idx1, rows1, sem1)
        _wait(idx0, rows0, sem0)
        _mac_rows(rows0)
        i2, j2 = _adv(i1, j1)
        _build(i2, j2, idx0)
        _issue(i2, j2, idx0, rows0, sem0)
        _wait(idx1, rows1, sem1)
        _mac_rows(rows1)
        return _adv(i2, j2)

    lax.fori_loop(0, (NPAIR - 1) // 2, _step, (jnp.int32(0), jnp.int32(2)))
    _wait(idx0, rows0, sem0)
    _mac_rows(rows0)

    w1copy.wait()
    for k in range(C // L):
        lin[pl.ds(k * L, L)] = zero

    def _lin_f(f, _):
        for k in range(C // L):
            idx = xv[f, pl.ds(k * L, L)]
            w = plsc.load_gather(w1v, [idx])
            plsc.addupdate(lin.at[pl.ds(k * L, L)], w)
        return 0
    lax.fori_loop(0, F, _lin_f, 0)

    w0vec = w0v[...]
    for g in range(C // L):
        t = jnp.zeros((L,), jnp.float32)
        for m in range(L):
            s = jnp.sum(acc[g * L + m], axis=0)
            t = jnp.where(lane == m, s, t)
        z = lin[pl.ds(g * L, L)] + t + w0vec
        res[pl.ds(g * L, L)] = 1.0 / (1.0 + jnp.exp(-z))
    pltpu.sync_copy(res, out_hbm.at[pl.ds(base, C)])


def kernel(x, v, w1, w0):
    import functools
    x = x.astype(jnp.int32)
    w1f = w1.reshape(H)
    w0v = jnp.broadcast_to(w0.astype(jnp.float32), (L,))
    vt1 = jnp.transpose(v[:FSPLIT], (0, 2, 1))
    vt2 = jnp.transpose(v[FSPLIT:], (0, 2, 1))

    repack = pl.kernel(
        functools.partial(_repack_body, nchunk=FSPLIT * NHB),
        out_type=jax.ShapeDtypeStruct((FSPLIT * H * K,), jnp.float32),
        mesh=_MESH,
        compiler_params=_PARAMS,
        scratch_types=[
            pltpu.VMEM((CE,), jnp.float32),
            pltpu.VMEM((CE,), jnp.float32),
            pltpu.VMEM((CE,), jnp.float32),
            pltpu.VMEM((CE,), jnp.float32),
            pltpu.SemaphoreType.DMA,
            pltpu.SemaphoreType.DMA,
            pltpu.SemaphoreType.DMA,
            pltpu.SemaphoreType.DMA,
        ],
    )
    vc1 = repack(vt1).reshape(FSPLIT * H, K)
    vc2 = repack(vt2).reshape(FSPLIT * H, K)

    ffm = pl.kernel(
        _ffm_body,
        out_type=jax.ShapeDtypeStruct((B,), jnp.float32),
        mesh=_MESH,
        compiler_params=_PARAMS,
        scratch_types=[
            pltpu.VMEM((F, C), jnp.int32),
            pltpu.VMEM((H,), jnp.float32),
            pltpu.VMEM((L,), jnp.float32),
            pltpu.VMEM((2, C), jnp.int32),
            pltpu.VMEM((2, C), jnp.int32),
            pltpu.VMEM((2, C, K), jnp.float32),
            pltpu.VMEM((2, C, K), jnp.float32),
            pltpu.VMEM((C, K), jnp.float32),
            pltpu.VMEM((C,), jnp.float32),
            pltpu.VMEM((C,), jnp.float32),
            pltpu.SemaphoreType.DMA,
            pltpu.SemaphoreType.DMA,
            pltpu.SemaphoreType.DMA,
        ],
    )
    return ffm(x, vc1, vc2, w1f, w0v)
